# baseline (speedup 1.0000x reference)
import functools

import jax
import jax.numpy as jnp
from jax import lax
from jax.experimental import pallas as pl
from jax.experimental.pallas import tpu as pltpu
from jax.experimental.pallas import tpu_sc as plsc

B = 16384
NS = 2
SB = B // NS
NW = 32
BPW = SB // NW
CH = 16
NCH = BPW // CH
BLK = 2048


def _sc_gather(user_rs, item_rs, guT, giT, muT, miT):
    mesh = plsc.VectorSubcoreMesh(core_axis_name="c", subcore_axis_name="s")

    @functools.partial(
        pl.kernel,
        mesh=mesh,
        compiler_params=pltpu.CompilerParams(needs_layout_passes=False),
        out_type=[
            jax.ShapeDtypeStruct((8, SB), jnp.float32),
            jax.ShapeDtypeStruct((8, SB), jnp.float32),
            jax.ShapeDtypeStruct((16, SB), jnp.float32),
            jax.ShapeDtypeStruct((16, SB), jnp.float32),
        ],
        scratch_types=[
            pltpu.VMEM((BPW,), jnp.int32),
            pltpu.VMEM((BPW,), jnp.int32),
            pltpu.VMEM((CH, 8, 128), jnp.float32),
            pltpu.VMEM((CH, 8, 128), jnp.float32),
            pltpu.VMEM((CH, 16, 128), jnp.float32),
            pltpu.VMEM((CH, 16, 128), jnp.float32),
            pltpu.VMEM((8, BPW), jnp.float32),
            pltpu.VMEM((8, BPW), jnp.float32),
            pltpu.VMEM((16, BPW), jnp.float32),
            pltpu.VMEM((16, BPW), jnp.float32),
            pltpu.SemaphoreType.DMA,
        ],
    )
    def k(user_h, item_h, gu_h, gi_h, mu_h, mi_h,
          gu_o, gi_o, mu_o, mi_o,
          vu, vi, tgu, tgi, tmu, tmi, bgu, bgi, bmu, bmi, sem):
        wid = lax.axis_index("s") * 2 + lax.axis_index("c")
        base = wid * BPW
        pltpu.sync_copy(user_h.at[wid], vu)
        pltpu.sync_copy(item_h.at[wid], vi)
        jvec = lax.iota(jnp.int32, 16)

        def chunk(c, carry):
            p0 = c * CH
            sl = pl.ds(p0, CH)
            uvals = vu[sl]
            ivals = vi[sl]
            descs = []
            for j in range(CH):
                tu = pl.multiple_of((uvals[j] >> 7) * 128, 128)
                ti = pl.multiple_of((ivals[j] >> 7) * 128, 128)
                descs.append(pltpu.async_copy(
                    gu_h.at[:, pl.ds(tu, 128)], tgu.at[j], sem))
                descs.append(pltpu.async_copy(
                    gi_h.at[:, pl.ds(ti, 128)], tgi.at[j], sem))
                descs.append(pltpu.async_copy(
                    mu_h.at[:, pl.ds(tu, 128)], tmu.at[j], sem))
                descs.append(pltpu.async_copy(
                    mi_h.at[:, pl.ds(ti, 128)], tmi.at[j], sem))
            for d in descs:
                d.wait()
            lu = uvals & 127
            li = ivals & 127
            for kk in range(8):
                kv = jnp.full((16,), kk, jnp.int32)
                bgu[kk, sl] = plsc.load_gather(tgu, [jvec, kv, lu])
                bgi[kk, sl] = plsc.load_gather(tgi, [jvec, kv, li])
            for kk in range(16):
                kv = jnp.full((16,), kk, jnp.int32)
                bmu[kk, sl] = plsc.load_gather(tmu, [jvec, kv, lu])
                bmi[kk, sl] = plsc.load_gather(tmi, [jvec, kv, li])
            return carry

        lax.fori_loop(0, NCH, chunk, 0)
        dst = pl.ds(base, BPW)
        pltpu.sync_copy(bgu, gu_o.at[:, dst])
        pltpu.sync_copy(bgi, gi_o.at[:, dst])
        pltpu.sync_copy(bmu, mu_o.at[:, dst])
        pltpu.sync_copy(bmi, mi_o.at[:, dst])

    return k(user_rs, item_rs, guT, giT, muT, miT)


def _tc_body(GU, GI, MU, MI, w1a, w1b, b1, w2, b2, wpg, wpm, bp, out):
    dot = functools.partial(jnp.dot, preferred_element_type=jnp.float32)
    h = jnp.maximum(dot(w1a[...], MU[...]) + dot(w1b[...], MI[...]) + b1[...],
                    0.0)
    m = jnp.maximum(dot(w2[...], h) + b2[...], 0.0)
    g = GU[...] * GI[...]
    val = dot(wpg[...], g) + dot(wpm[...], m) + bp[...]
    out[...] = jax.nn.sigmoid(val)


def _tc_dense(GU, GI, MU, MI, w1a, w1b, b1, w2, b2, wpg, wpm, bp):
    grid = SB // BLK
    return pl.pallas_call(
        _tc_body,
        grid=(grid,),
        in_specs=[
            pl.BlockSpec((8, BLK), lambda i: (0, i)),
            pl.BlockSpec((8, BLK), lambda i: (0, i)),
            pl.BlockSpec((16, BLK), lambda i: (0, i)),
            pl.BlockSpec((16, BLK), lambda i: (0, i)),
            pl.BlockSpec((16, 16), lambda i: (0, 0)),
            pl.BlockSpec((16, 16), lambda i: (0, 0)),
            pl.BlockSpec((16, 1), lambda i: (0, 0)),
            pl.BlockSpec((8, 16), lambda i: (0, 0)),
            pl.BlockSpec((8, 1), lambda i: (0, 0)),
            pl.BlockSpec((1, 8), lambda i: (0, 0)),
            pl.BlockSpec((1, 8), lambda i: (0, 0)),
            pl.BlockSpec((1, 1), lambda i: (0, 0)),
        ],
        out_specs=pl.BlockSpec((1, BLK), lambda i: (0, i)),
        out_shape=jax.ShapeDtypeStruct((1, SB), jnp.float32),
    )(GU, GI, MU, MI, w1a, w1b, b1, w2, b2, wpg, wpm, bp)


def kernel(user, item, gmf_user_emb, gmf_item_emb, mlp_user_emb, mlp_item_emb,
           W1, b1, W2, b2, Wp, bp):
    user_rs = user.astype(jnp.int32).reshape(NS, NW, BPW)
    item_rs = item.astype(jnp.int32).reshape(NS, NW, BPW)
    guT, giT = gmf_user_emb.T, gmf_item_emb.T
    muT, miT = mlp_user_emb.T, mlp_item_emb.T
    wargs = (W1[:16].T, W1[16:].T, b1.reshape(16, 1),
             W2.T, b2.reshape(8, 1),
             Wp[:8].T, Wp[8:].T, bp.reshape(1, 1))
    outs = []
    for s in range(NS):
        GU, GI, MU, MI = _sc_gather(
            user_rs[s], item_rs[s], guT, giT, muT, miT)
        outs.append(_tc_dense(GU, GI, MU, MI, *wargs))
    return jnp.concatenate(outs, axis=1).reshape(-1)


# speedup vs baseline: 1.0998x; 1.0998x over previous
import functools

import jax
import jax.numpy as jnp
from jax import lax
from jax.experimental import pallas as pl
from jax.experimental.pallas import tpu as pltpu
from jax.experimental.pallas import tpu_sc as plsc

B = 16384
NW = 32
BPW = B // NW
CH = 16
NCH = BPW // CH

_W2OFF = 512
_B1OFF = 640
_B2OFF = 656
_WPOFF = 672
WLEN = 688


def _neumf_sc(user_rs, item_rs, guT, giT, muT, miT, wcat):
    mesh = plsc.VectorSubcoreMesh(core_axis_name="c", subcore_axis_name="s")

    @functools.partial(
        pl.kernel,
        mesh=mesh,
        compiler_params=pltpu.CompilerParams(needs_layout_passes=False),
        out_type=jax.ShapeDtypeStruct((B,), jnp.float32),
        scratch_types=[
            pltpu.VMEM((BPW,), jnp.int32),
            pltpu.VMEM((BPW,), jnp.int32),
            pltpu.VMEM((WLEN,), jnp.float32),
            pltpu.VMEM((CH, 8, 128), jnp.float32),
            pltpu.VMEM((CH, 8, 128), jnp.float32),
            pltpu.VMEM((CH, 16, 128), jnp.float32),
            pltpu.VMEM((CH, 16, 128), jnp.float32),
            pltpu.VMEM((BPW,), jnp.float32),
            pltpu.SemaphoreType.DMA,
        ],
    )
    def k(user_h, item_h, gu_h, gi_h, mu_h, mi_h, w_h, out_o,
          vu, vi, wv, tgu, tgi, tmu, tmi, obuf, sem):
        wid = lax.axis_index("s") * 2 + lax.axis_index("c")
        base = wid * BPW
        pltpu.sync_copy(user_h.at[wid], vu)
        pltpu.sync_copy(item_h.at[wid], vi)
        pltpu.sync_copy(w_h, wv)
        jvec = lax.iota(jnp.int32, 16)

        w1v = [wv[pl.ds(16 * m, 16)] for m in range(32)]
        w2v = [wv[pl.ds(_W2OFF + 16 * j, 16)] for j in range(8)]
        b1v = wv[pl.ds(_B1OFF, 16)]
        b2wp = wv[pl.ds(_B2OFF, 16)]
        wpbp = wv[pl.ds(_WPOFF, 16)]

        def enqueue(c):
            sl = pl.ds(c * CH, CH)
            uvals = vu[sl]
            ivals = vi[sl]
            for j in range(CH):
                tu = pl.multiple_of((uvals[j] >> 7) * 128, 128)
                ti = pl.multiple_of((ivals[j] >> 7) * 128, 128)
                pltpu.async_copy(gu_h.at[:, pl.ds(tu, 128)], tgu.at[j], sem)
                pltpu.async_copy(gi_h.at[:, pl.ds(ti, 128)], tgi.at[j], sem)
                pltpu.async_copy(mu_h.at[:, pl.ds(tu, 128)], tmu.at[j], sem)
                pltpu.async_copy(mi_h.at[:, pl.ds(ti, 128)], tmi.at[j], sem)

        enqueue(0)

        def chunk(c, carry):
            src128 = pl.ds(0, 128)
            for j in range(CH):
                pltpu.make_async_copy(
                    gu_h.at[:, src128], tgu.at[j], sem).wait()
                pltpu.make_async_copy(
                    gi_h.at[:, src128], tgi.at[j], sem).wait()
                pltpu.make_async_copy(
                    mu_h.at[:, src128], tmu.at[j], sem).wait()
                pltpu.make_async_copy(
                    mi_h.at[:, src128], tmi.at[j], sem).wait()

            sl = pl.ds(c * CH, CH)
            lu = vu[sl] & 127
            li = vi[sl] & 127
            kvs = [jnp.full((16,), kk, jnp.int32) for kk in range(16)]
            gu_r = [plsc.load_gather(tgu, [jvec, kvs[kk], lu])
                    for kk in range(8)]
            gi_r = [plsc.load_gather(tgi, [jvec, kvs[kk], li])
                    for kk in range(8)]
            mu_r = [plsc.load_gather(tmu, [jvec, kvs[kk], lu])
                    for kk in range(16)]
            mi_r = [plsc.load_gather(tmi, [jvec, kvs[kk], li])
                    for kk in range(16)]

            @pl.when(c + 1 < NCH)
            def _():
                enqueue(c + 1)

            hs = []
            for j in range(16):
                acc = jnp.full((16,), b1v[j], jnp.float32)
                wa, wb = w1v[2 * j], w1v[2 * j + 1]
                for kk in range(16):
                    acc = acc + mu_r[kk] * wa[kk]
                    acc = acc + mi_r[kk] * wb[kk]
                hs.append(jnp.maximum(acc, 0.0))
            val = jnp.full((16,), wpbp[8], jnp.float32)
            for j in range(8):
                acc = jnp.full((16,), b2wp[j], jnp.float32)
                w2j = w2v[j]
                for kk in range(16):
                    acc = acc + hs[kk] * w2j[kk]
                m_j = jnp.maximum(acc, 0.0)
                g_j = gu_r[j] * gi_r[j]
                val = val + g_j * b2wp[8 + j] + m_j * wpbp[j]
            obuf[sl] = 1.0 / (1.0 + jnp.exp(-val))
            return carry

        lax.fori_loop(0, NCH, chunk, 0)
        pltpu.sync_copy(obuf, out_o.at[pl.ds(base, BPW)])

    return k(user_rs, item_rs, guT, giT, muT, miT, wcat)


def kernel(user, item, gmf_user_emb, gmf_item_emb, mlp_user_emb, mlp_item_emb,
           W1, b1, W2, b2, Wp, bp):
    user_rs = user.astype(jnp.int32).reshape(NW, BPW)
    item_rs = item.astype(jnp.int32).reshape(NW, BPW)
    wcat = jnp.concatenate([
        W1.T.reshape(-1), W2.T.reshape(-1), b1, b2, Wp.reshape(-1), bp,
        jnp.zeros((WLEN - 681,), jnp.float32)])
    return _neumf_sc(user_rs, item_rs,
                     gmf_user_emb.T, gmf_item_emb.T,
                     mlp_user_emb.T, mlp_item_emb.T, wcat)


# speedup vs baseline: 1.1067x; 1.0063x over previous
import functools

import jax
import jax.numpy as jnp
from jax import lax
from jax.experimental import pallas as pl
from jax.experimental.pallas import tpu as pltpu
from jax.experimental.pallas import tpu_sc as plsc

B = 16384
NW = 32
BPW = B // NW
CH = 16
NCH = BPW // CH

def _neumf_sc(user, item, guT, giT, muT, miT, W1T, b1, W2T, b2, WpT, bp):
    mesh = plsc.VectorSubcoreMesh(core_axis_name="c", subcore_axis_name="s")

    @functools.partial(
        pl.kernel,
        mesh=mesh,
        compiler_params=pltpu.CompilerParams(needs_layout_passes=False),
        out_type=jax.ShapeDtypeStruct((B,), jnp.float32),
        scratch_types=[
            pltpu.VMEM((BPW,), jnp.int32),
            pltpu.VMEM((BPW,), jnp.int32),
            pltpu.VMEM((16, 32), jnp.float32),
            pltpu.VMEM((8, 16), jnp.float32),
            pltpu.VMEM((16,), jnp.float32),
            pltpu.VMEM((16,), jnp.float32),
            pltpu.VMEM((1, 16), jnp.float32),
            pltpu.VMEM((16,), jnp.float32),
            pltpu.VMEM((CH, 8, 128), jnp.float32),
            pltpu.VMEM((CH, 8, 128), jnp.float32),
            pltpu.VMEM((CH, 16, 128), jnp.float32),
            pltpu.VMEM((CH, 16, 128), jnp.float32),
            pltpu.VMEM((BPW,), jnp.float32),
            pltpu.SemaphoreType.DMA,
        ],
    )
    def k(user_h, item_h, gu_h, gi_h, mu_h, mi_h,
          w1_h, b1_h, w2_h, b2_h, wp_h, bp_h, out_o,
          vu, vi, w1s, w2s, b1s, b2s, wps, bps, tgu, tgi, tmu, tmi,
          obuf, sem):
        wid = lax.axis_index("s") * 2 + lax.axis_index("c")
        base = wid * BPW
        pltpu.sync_copy(user_h.at[pl.ds(base, BPW)], vu)
        pltpu.sync_copy(item_h.at[pl.ds(base, BPW)], vi)
        pltpu.sync_copy(w1_h, w1s)
        pltpu.sync_copy(w2_h, w2s)
        pltpu.sync_copy(b1_h, b1s)
        pltpu.sync_copy(b2_h, b2s.at[pl.ds(0, 8)])
        pltpu.sync_copy(wp_h, wps)
        pltpu.sync_copy(bp_h, bps.at[pl.ds(0, 1)])
        jvec = lax.iota(jnp.int32, 16)

        w1v = []
        for j in range(16):
            w1v.append(w1s[j, pl.ds(0, 16)])
            w1v.append(w1s[j, pl.ds(16, 16)])
        w2v = [w2s[j, pl.ds(0, 16)] for j in range(8)]
        b1v = b1s[pl.ds(0, 16)]
        b2v = b2s[pl.ds(0, 16)]
        wpv = wps[0, pl.ds(0, 16)]
        bpv = bps[pl.ds(0, 16)]

        def enqueue(c):
            sl = pl.ds(c * CH, CH)
            uvals = vu[sl]
            ivals = vi[sl]
            for j in range(CH):
                tu = pl.multiple_of((uvals[j] >> 7) * 128, 128)
                ti = pl.multiple_of((ivals[j] >> 7) * 128, 128)
                pltpu.async_copy(gu_h.at[:, pl.ds(tu, 128)], tgu.at[j], sem)
                pltpu.async_copy(gi_h.at[:, pl.ds(ti, 128)], tgi.at[j], sem)
                pltpu.async_copy(mu_h.at[:, pl.ds(tu, 128)], tmu.at[j], sem)
                pltpu.async_copy(mi_h.at[:, pl.ds(ti, 128)], tmi.at[j], sem)

        enqueue(0)

        def chunk(c, carry):
            src128 = pl.ds(0, 128)
            for j in range(CH):
                pltpu.make_async_copy(
                    gu_h.at[:, src128], tgu.at[j], sem).wait()
                pltpu.make_async_copy(
                    gi_h.at[:, src128], tgi.at[j], sem).wait()
                pltpu.make_async_copy(
                    mu_h.at[:, src128], tmu.at[j], sem).wait()
                pltpu.make_async_copy(
                    mi_h.at[:, src128], tmi.at[j], sem).wait()

            sl = pl.ds(c * CH, CH)
            lu = vu[sl] & 127
            li = vi[sl] & 127
            kvs = [jnp.full((16,), kk, jnp.int32) for kk in range(16)]
            gu_r = [plsc.load_gather(tgu, [jvec, kvs[kk], lu])
                    for kk in range(8)]
            gi_r = [plsc.load_gather(tgi, [jvec, kvs[kk], li])
                    for kk in range(8)]
            mu_r = [plsc.load_gather(tmu, [jvec, kvs[kk], lu])
                    for kk in range(16)]
            mi_r = [plsc.load_gather(tmi, [jvec, kvs[kk], li])
                    for kk in range(16)]

            @pl.when(c + 1 < NCH)
            def _():
                enqueue(c + 1)

            hs = []
            for j in range(16):
                acc = jnp.full((16,), b1v[j], jnp.float32)
                wa, wb = w1v[2 * j], w1v[2 * j + 1]
                for kk in range(16):
                    acc = acc + mu_r[kk] * wa[kk]
                    acc = acc + mi_r[kk] * wb[kk]
                hs.append(jnp.maximum(acc, 0.0))
            val = jnp.full((16,), bpv[0], jnp.float32)
            for j in range(8):
                acc = jnp.full((16,), b2v[j], jnp.float32)
                w2j = w2v[j]
                for kk in range(16):
                    acc = acc + hs[kk] * w2j[kk]
                m_j = jnp.maximum(acc, 0.0)
                g_j = gu_r[j] * gi_r[j]
                val = val + g_j * wpv[j] + m_j * wpv[8 + j]
            obuf[sl] = 1.0 / (1.0 + jnp.exp(-val))
            return carry

        lax.fori_loop(0, NCH, chunk, 0)
        pltpu.sync_copy(obuf, out_o.at[pl.ds(base, BPW)])

    return k(user, item, guT, giT, muT, miT, W1T, b1, W2T, b2, WpT, bp)


def kernel(user, item, gmf_user_emb, gmf_item_emb, mlp_user_emb, mlp_item_emb,
           W1, b1, W2, b2, Wp, bp):
    return _neumf_sc(user.astype(jnp.int32), item.astype(jnp.int32),
                     gmf_user_emb.T, gmf_item_emb.T,
                     mlp_user_emb.T, mlp_item_emb.T,
                     W1.T, b1, W2.T, b2, Wp.T, bp)
